# baseline (speedup 1.0000x reference)
import functools

import jax
import jax.numpy as jnp
from jax import lax
from jax.experimental import pallas as pl
from jax.experimental.pallas import tpu as pltpu
from jax.experimental.pallas import tpu_sc as plsc

_NW = 32
_CW = 128
_BB = 256


def _gelu(x):
    return 0.5 * x * (1.0 + lax.erf(x * 0.7071067811865476))


def _sc_gather(table, ids3):
    NW, CH, CW = ids3.shape
    d = table.shape[1]
    n = NW * CH * CW
    mesh = plsc.VectorSubcoreMesh(core_axis_name="c", subcore_axis_name="s")

    NB = 4

    @functools.partial(
        pl.kernel,
        mesh=mesh,
        out_type=jax.ShapeDtypeStruct((n, d), jnp.float32),
        scratch_types=(
            [pltpu.VMEM((CH, CW), jnp.int32)]
            + [pltpu.VMEM((CW, d), jnp.float32) for _ in range(NB)]
            + [pltpu.SemaphoreType.DMA for _ in range(2 * NB)]
        ),
    )
    def gather_kernel(table_hbm, idx_hbm, out_hbm, idx_v, *rest):
        bufs = rest[:NB]
        gsem = rest[NB:2 * NB]
        ssem = rest[2 * NB:]
        wid = lax.axis_index("s") * 2 + lax.axis_index("c")
        pltpu.sync_copy(idx_hbm.at[wid], idx_v)
        g = [pltpu.async_copy(table_hbm.at[idx_v.at[c]], bufs[c], gsem[c])
             for c in range(min(NB, CH))]
        st = [None] * NB
        for c in range(CH):
            b = c % NB
            g[b].wait()
            st[b] = pltpu.async_copy(
                bufs[b], out_hbm.at[pl.ds((wid * CH + c) * CW, CW)], ssem[b])
            if c + NB < CH:
                st[b].wait()
                g[b] = pltpu.async_copy(
                    table_hbm.at[idx_v.at[c + NB]], bufs[b], gsem[b])
        for c in range(max(0, CH - NB), CH):
            st[c % NB].wait()

    return gather_kernel(table, ids3)


def _tc_body(actor_ref, career_ref, genre_ref, role_ref, rtab_ref,
             Win_ref, bin_ref, cW2_ref, cb2_ref, fW1b_ref, fW1acg_ref,
             fb1_ref, fW2_ref, fb2_ref, out_ref):
    f32 = jnp.float32
    seq, bb, hdim = out_ref.shape
    t = seq * bb
    q = cW2_ref.shape[0]
    nr = rtab_ref.shape[0]
    career2 = career_ref[...].reshape(t, career_ref.shape[2])
    genre2 = genre_ref[...].reshape(t, genre_ref.shape[2])
    cg2 = jnp.concatenate([career2, genre2], axis=-1)
    ri3 = lax.broadcast_in_dim(role_ref[...], (seq, bb, nr), (0, 1))
    oh3 = jnp.where(ri3 == lax.broadcasted_iota(jnp.int32, (seq, bb, nr), 2),
                    1.0, 0.0).astype(f32)
    onehot = oh3.reshape(t, nr)
    u = jnp.dot(cg2, Win_ref[...], preferred_element_type=f32) + bin_ref[...]
    c1 = _gelu(u[:, :q])
    career_emb = jnp.dot(c1, cW2_ref[...], preferred_element_type=f32) + cb2_ref[...]
    genre_emb = u[:, q:]
    rT = jnp.dot(rtab_ref[...], fW1b_ref[...], preferred_element_type=f32)
    W1 = jnp.concatenate([fW1acg_ref[...], rT], axis=0)
    actor2 = actor_ref[...].reshape(t, actor_ref.shape[2])
    comb = jnp.concatenate([actor2, career_emb, genre_emb, onehot], axis=-1)
    h = _gelu(jnp.dot(comb, W1, preferred_element_type=f32) + fb1_ref[...])
    out = jnp.dot(h, fW2_ref[...], preferred_element_type=f32) + fb2_ref[...]
    out_ref[...] = out.reshape(seq, bb, hdim)


def _tc_fused(actor_emb, career, genre, roles, role_table, Win, bin_, cW2,
              cb2, fW1b, fW1acg, fb1, fW2, fb2, bsz_total, boff,
              out_full=None):
    seq, bsz, _ = career.shape
    d = actor_emb.shape[2]
    hdim = fW2.shape[1]
    bb = _BB

    def full(a):
        return pl.BlockSpec(a.shape, lambda i: tuple(0 for _ in a.shape))

    in_specs = [
        pl.BlockSpec((seq, bb, d), lambda i: (0, i, 0)),
        pl.BlockSpec((seq, bb, career.shape[2]), lambda i: (0, i, 0)),
        pl.BlockSpec((seq, bb, genre.shape[2]), lambda i: (0, i, 0)),
        pl.BlockSpec((seq, bb), lambda i: (0, i)),
        full(role_table), full(Win), full(bin_), full(cW2), full(cb2),
        full(fW1b), full(fW1acg), full(fb1), full(fW2), full(fb2),
    ]
    args = (actor_emb, career, genre, roles, role_table, Win, bin_, cW2,
            cb2, fW1b, fW1acg, fb1, fW2, fb2)
    body = _tc_body
    aliases = {}
    if out_full is not None:
        def body(dummy_ref, *refs):
            _tc_body(*refs)
        in_specs = [pl.BlockSpec(memory_space=pl.ANY)] + in_specs
        args = (out_full,) + args
        aliases = {0: 0}
    return pl.pallas_call(
        body,
        grid=(bsz // bb,),
        in_specs=in_specs,
        out_specs=pl.BlockSpec((seq, bb, hdim), lambda i: (0, boff + i, 0)),
        out_shape=jax.ShapeDtypeStruct((seq, bsz_total, hdim), jnp.float32),
        input_output_aliases=aliases,
    )(*args)


def kernel(actor_ids, role_types, career_features, genre_distribution,
           actor_table, role_table, cW1, cb1, cW2, cb2, gW, gb,
           fW1, fb1, fW2, fb2):
    bsz, seq = actor_ids.shape
    n = bsz * seq
    d = actor_table.shape[1]
    q = cW1.shape[1]
    fW1a = fW1[:d]
    fW1b = fW1[d:d + q]
    fW1c = fW1[d + q:d + 2 * q]
    fW1g = fW1[d + 2 * q:]
    nc = cW1.shape[0]
    ng = gW.shape[0]
    Win = jnp.zeros((nc + ng, 2 * q), dtype=jnp.float32)
    Win = Win.at[:nc, :q].set(cW1).at[nc:, q:].set(gW)
    bin_ = jnp.concatenate([cb1, gb]).reshape(1, -1)
    fW1acg = jnp.concatenate([fW1a, fW1c, fW1g], axis=0)
    careerT = career_features.transpose(1, 0, 2)
    genreT = genre_distribution.transpose(1, 0, 2)
    rolesT = role_types.transpose(1, 0).astype(jnp.int32)
    nchunks = 2
    cw = bsz // nchunks
    out = None
    for c in range(nchunks):
        lo, hi = c * cw, (c + 1) * cw
        ids3 = (actor_ids[lo:hi].transpose(1, 0).astype(jnp.int32)
                .reshape(_NW, (seq * cw) // (_NW * _CW), _CW))
        a_emb = _sc_gather(actor_table, ids3).reshape(seq, cw, d)
        out = _tc_fused(
            a_emb, careerT[:, lo:hi], genreT[:, lo:hi], rolesT[:, lo:hi],
            role_table, Win, bin_, cW2, cb2.reshape(1, -1), fW1b, fW1acg,
            fb1.reshape(1, -1), fW2, fb2.reshape(1, -1),
            bsz_total=bsz, boff=c * (cw // _BB), out_full=out)
    return out.transpose(1, 0, 2)


# speedup vs baseline: 1.3399x; 1.3399x over previous
import functools

import jax
import jax.numpy as jnp
from jax import lax
from jax.experimental import pallas as pl
from jax.experimental.pallas import tpu as pltpu
from jax.experimental.pallas import tpu_sc as plsc

_NW = 32
_CW = 128
_BB = 256


def _gelu(x):
    return 0.5 * x * (1.0 + lax.erf(x * 0.7071067811865476))


def _sc_gather(table, ids3):
    NW, CH, CW = ids3.shape
    d = table.shape[1]
    n = NW * CH * CW
    mesh = plsc.VectorSubcoreMesh(core_axis_name="c", subcore_axis_name="s")

    NB = 4

    @functools.partial(
        pl.kernel,
        mesh=mesh,
        out_type=jax.ShapeDtypeStruct((n, d), jnp.float32),
        scratch_types=(
            [pltpu.VMEM((CH, CW), jnp.int32)]
            + [pltpu.VMEM((CW, d), jnp.float32) for _ in range(NB)]
            + [pltpu.SemaphoreType.DMA for _ in range(2 * NB)]
        ),
    )
    def gather_kernel(table_hbm, idx_hbm, out_hbm, idx_v, *rest):
        bufs = rest[:NB]
        gsem = rest[NB:2 * NB]
        ssem = rest[2 * NB:]
        wid = lax.axis_index("s") * 2 + lax.axis_index("c")
        pltpu.sync_copy(idx_hbm.at[wid], idx_v)
        g = [pltpu.async_copy(table_hbm.at[idx_v.at[c]], bufs[c], gsem[c])
             for c in range(min(NB, CH))]
        st = [None] * NB
        for c in range(CH):
            b = c % NB
            g[b].wait()
            st[b] = pltpu.async_copy(
                bufs[b], out_hbm.at[pl.ds((wid * CH + c) * CW, CW)], ssem[b])
            if c + NB < CH:
                st[b].wait()
                g[b] = pltpu.async_copy(
                    table_hbm.at[idx_v.at[c + NB]], bufs[b], gsem[b])
        for c in range(max(0, CH - NB), CH):
            st[c % NB].wait()

    return gather_kernel(table, ids3)


def _tc_body(actor_ref, career_ref, genre_ref, role_ref, rtab_ref,
             Win_ref, bin_ref, cW2_ref, cb2_ref, fW1b_ref, fW1acg_ref,
             fb1_ref, fW2_ref, fb2_ref, out_ref):
    f32 = jnp.float32
    seq, bb, hdim = out_ref.shape
    t = seq * bb
    q = cW2_ref.shape[0]
    nr = rtab_ref.shape[0]
    career2 = career_ref[...].reshape(t, career_ref.shape[2])
    genre2 = genre_ref[...].reshape(t, genre_ref.shape[2])
    cg2 = jnp.concatenate([career2, genre2], axis=-1)
    ri3 = lax.broadcast_in_dim(role_ref[...], (seq, bb, nr), (0, 1))
    oh3 = jnp.where(ri3 == lax.broadcasted_iota(jnp.int32, (seq, bb, nr), 2),
                    1.0, 0.0).astype(f32)
    onehot = oh3.reshape(t, nr)
    u = jnp.dot(cg2, Win_ref[...], preferred_element_type=f32) + bin_ref[...]
    c1 = _gelu(u[:, :q])
    career_emb = jnp.dot(c1, cW2_ref[...], preferred_element_type=f32) + cb2_ref[...]
    genre_emb = u[:, q:]
    rT = jnp.dot(rtab_ref[...], fW1b_ref[...], preferred_element_type=f32)
    W1 = jnp.concatenate([fW1acg_ref[...], rT], axis=0)
    actor2 = actor_ref[...].reshape(t, actor_ref.shape[2])
    comb = jnp.concatenate([actor2, career_emb, genre_emb, onehot], axis=-1)
    h = _gelu(jnp.dot(comb, W1, preferred_element_type=f32) + fb1_ref[...])
    out = jnp.dot(h, fW2_ref[...], preferred_element_type=f32) + fb2_ref[...]
    out_ref[...] = out.reshape(seq, bb, hdim)


def _tc_fused(actor_emb, career, genre, roles, role_table, Win, bin_, cW2,
              cb2, fW1b, fW1acg, fb1, fW2, fb2):
    seq, bsz, _ = career.shape
    d = actor_emb.shape[2]
    hdim = fW2.shape[1]
    bb = _BB

    def full(a):
        return pl.BlockSpec(a.shape, lambda i: tuple(0 for _ in a.shape))

    in_specs = [
        pl.BlockSpec((seq, bb, d), lambda i: (0, i, 0)),
        pl.BlockSpec((seq, bb, career.shape[2]), lambda i: (0, i, 0)),
        pl.BlockSpec((seq, bb, genre.shape[2]), lambda i: (0, i, 0)),
        pl.BlockSpec((seq, bb), lambda i: (0, i)),
        full(role_table), full(Win), full(bin_), full(cW2), full(cb2),
        full(fW1b), full(fW1acg), full(fb1), full(fW2), full(fb2),
    ]
    return pl.pallas_call(
        _tc_body,
        grid=(bsz // bb,),
        in_specs=in_specs,
        out_specs=pl.BlockSpec((seq, bb, hdim), lambda i: (0, i, 0)),
        out_shape=jax.ShapeDtypeStruct((seq, bsz, hdim), jnp.float32),
    )(actor_emb, career, genre, roles, role_table, Win, bin_, cW2, cb2,
      fW1b, fW1acg, fb1, fW2, fb2)


def kernel(actor_ids, role_types, career_features, genre_distribution,
           actor_table, role_table, cW1, cb1, cW2, cb2, gW, gb,
           fW1, fb1, fW2, fb2):
    bsz, seq = actor_ids.shape
    n = bsz * seq
    d = actor_table.shape[1]
    q = cW1.shape[1]
    ids3 = (actor_ids.transpose(1, 0).astype(jnp.int32)
            .reshape(_NW, n // (_NW * _CW), _CW))
    actor_emb = _sc_gather(actor_table, ids3).reshape(seq, bsz, d)
    fW1a = fW1[:d]
    fW1b = fW1[d:d + q]
    fW1c = fW1[d + q:d + 2 * q]
    fW1g = fW1[d + 2 * q:]
    nc = cW1.shape[0]
    ng = gW.shape[0]
    Win = jnp.zeros((nc + ng, 2 * q), dtype=jnp.float32)
    Win = Win.at[:nc, :q].set(cW1).at[nc:, q:].set(gW)
    bin_ = jnp.concatenate([cb1, gb]).reshape(1, -1)
    fW1acg = jnp.concatenate([fW1a, fW1c, fW1g], axis=0)
    out = _tc_fused(
        actor_emb, career_features.transpose(1, 0, 2),
        genre_distribution.transpose(1, 0, 2),
        role_types.transpose(1, 0).astype(jnp.int32), role_table, Win, bin_,
        cW2, cb2.reshape(1, -1), fW1b, fW1acg, fb1.reshape(1, -1), fW2,
        fb2.reshape(1, -1))
    return out.transpose(1, 0, 2)
